# baseline (speedup 1.0000x reference)
import jax
import jax.numpy as jnp
from jax import lax
from jax.experimental import pallas as pl
from jax.experimental.pallas import tpu as pltpu

_S_TOPK = 64
_TAU = 0.1
_ETA = 0.5
_N_SOFT_LAYERS = 3
_L = 128


def _cmpx(x, d, keep_max):
    size = x.shape[-1]
    lower = (lax.broadcasted_iota(jnp.int32, x.shape, x.ndim - 1) & d) == 0
    partner = jnp.where(lower,
                        pltpu.roll(x, size - d, x.ndim - 1),
                        pltpu.roll(x, d, x.ndim - 1))
    return jnp.where(keep_max, jnp.maximum(x, partner),
                     jnp.minimum(x, partner))


def _asc_flag(shape):
    c = shape[1]
    if c == 1:
        return jnp.zeros(shape, dtype=jnp.bool_)
    sub = lax.broadcasted_iota(jnp.int32, shape, 1)
    return sub >= (c // 2)


def _sort128(x):
    asc = _asc_flag(x.shape)
    i = lax.broadcasted_iota(jnp.int32, x.shape, x.ndim - 1)
    for p in range(1, 8):
        block_desc = ((i >> p) & 1) == 0
        for q in range(p - 1, -1, -1):
            d = 1 << q
            keep_max = (((i & d) == 0) == block_desc) != asc
            x = _cmpx(x, d, keep_max)
    return x


def _clean128(x):
    asc = _asc_flag(x.shape)
    i = lax.broadcasted_iota(jnp.int32, x.shape, x.ndim - 1)
    for q in range(6, -1, -1):
        d = 1 << q
        keep_max = ((i & d) == 0) != asc
        x = _cmpx(x, d, keep_max)
    return x


def _top128(s3):
    x = _sort128(s3)
    while x.shape[1] > 1:
        h = x.shape[1] // 2
        x = _clean128(jnp.maximum(x[:, :h], x[:, h:]))
    return x


def _body(y_ref, a_ref, w_ref, o_ref):
    batch, m = y_ref.shape
    _, n = a_ref.shape
    f32 = jnp.float32

    a = a_ref[...]
    gram = lax.dot_general(a, a, (((0,), (0,)), ((), ())),
                           preferred_element_type=f32)
    ii = lax.broadcasted_iota(jnp.int32, (n, n), 0)
    jj = lax.broadcasted_iota(jnp.int32, (n, n), 1)
    mm = jnp.where(ii == jj, f32(1.0), f32(0.0)) - f32(_ETA) * gram

    y = y_ref[...]
    bt = f32(_ETA) * lax.dot_general(y, a, (((1,), (0,)), ((), ())),
                                     preferred_element_type=f32)
    w = w_ref[...]
    inv_tau = f32(1.0 / _TAU)

    xt = jnp.zeros((batch, n), dtype=f32)
    for layer in range(_N_SOFT_LAYERS):
        if layer == 0:
            ht = bt
        else:
            ht = bt + lax.dot_general(xt, mm, (((1,), (0,)), ((), ())),
                                      preferred_element_type=f32)
        s0 = jnp.abs(ht * w)

        t = _top128(s0.reshape(batch, n // _L, _L))
        t = t.reshape(batch, _L)[:, :_S_TOPK]
        t63 = t[:, _S_TOPK - 1:_S_TOPK]

        g = jnp.exp((t63 - t) * inv_tau)
        u = jnp.exp((s0 - t63) * inv_tau)
        below = s0 < t63
        v = jnp.sum(jnp.where(below, u, f32(0.0)), axis=1, keepdims=True)
        n_ge = jnp.sum(jnp.where(below, f32(0.0), f32(1.0)), axis=1,
                       keepdims=True)

        p = jnp.zeros((batch, _S_TOPK), dtype=f32)
        for r in range(_S_TOPK):
            p = p + jnp.exp(-jnp.abs(t - t[:, r:r + 1]) * inv_tau)
        z = p + g * (v + n_ge - f32(_S_TOPK))
        zinv = f32(1.0) / z
        k_below = jnp.sum(zinv * g, axis=1, keepdims=True)

        mval = jnp.zeros((batch, _S_TOPK), dtype=f32)
        for r in range(_S_TOPK):
            row = jnp.exp(-jnp.abs(t - t[:, r:r + 1]) * inv_tau)
            mval = mval + row * zinv[:, r:r + 1]

        acc = jnp.zeros((batch, n), f32) + mval[:, 0:1]
        for r in range(_S_TOPK - 1):
            dm = mval[:, r + 1:r + 2] - mval[:, r:r + 1]
            acc = acc + jnp.where(s0 < t[:, r:r + 1], dm, f32(0.0))
        mask = jnp.where(below, u * k_below, acc)
        xt = mask * ht

    o_ref[...] = xt


def kernel(Y, A, W):
    batch, _ = Y.shape
    _, n = A.shape
    return pl.pallas_call(
        _body,
        out_shape=jax.ShapeDtypeStruct((batch, n), jnp.float32),
    )(Y, A, W.reshape(1, n))


# speedup vs baseline: 2.2947x; 2.2947x over previous
import jax
import jax.numpy as jnp
from jax import lax
from jax.experimental import pallas as pl
from jax.experimental.pallas import tpu as pltpu

_S_TOPK = 64
_TAU = 0.1
_ETA = 0.5
_N_SOFT_LAYERS = 3
_L = 128


def _cmpx(x, d, keep_max):
    size = x.shape[-1]
    lower = (lax.broadcasted_iota(jnp.int32, x.shape, x.ndim - 1) & d) == 0
    partner = jnp.where(lower,
                        pltpu.roll(x, size - d, x.ndim - 1),
                        pltpu.roll(x, d, x.ndim - 1))
    return jnp.where(keep_max, jnp.maximum(x, partner),
                     jnp.minimum(x, partner))


def _asc_flag(shape):
    c = shape[1]
    if c == 1:
        return jnp.zeros(shape, dtype=jnp.bool_)
    sub = lax.broadcasted_iota(jnp.int32, shape, 1)
    return sub >= (c // 2)


def _sort128(x):
    asc = _asc_flag(x.shape)
    i = lax.broadcasted_iota(jnp.int32, x.shape, x.ndim - 1)
    for p in range(1, 8):
        block_desc = ((i >> p) & 1) == 0
        for q in range(p - 1, -1, -1):
            d = 1 << q
            keep_max = (((i & d) == 0) == block_desc) != asc
            x = _cmpx(x, d, keep_max)
    return x


def _clean128(x):
    asc = _asc_flag(x.shape)
    i = lax.broadcasted_iota(jnp.int32, x.shape, x.ndim - 1)
    for q in range(6, -1, -1):
        d = 1 << q
        keep_max = ((i & d) == 0) != asc
        x = _cmpx(x, d, keep_max)
    return x


def _top128(s3):
    x = _sort128(s3)
    while x.shape[1] > 1:
        h = x.shape[1] // 2
        x = _clean128(jnp.maximum(x[:, :h], x[:, h:]))
    return x


def _scan_sum(x, reverse):
    size = x.shape[-1]
    lane = lax.broadcasted_iota(jnp.int32, x.shape, x.ndim - 1)
    q = 1
    while q < size:
        if reverse:
            x = x + jnp.where(lane < size - q,
                              pltpu.roll(x, size - q, x.ndim - 1), 0.0)
        else:
            x = x + jnp.where(lane >= q, pltpu.roll(x, q, x.ndim - 1), 0.0)
        q *= 2
    return x


def _shift_right(x, lane):
    size = x.shape[-1]
    return jnp.where(lane >= 1, pltpu.roll(x, 1, x.ndim - 1), 0.0)


def _shift_left(x, lane):
    size = x.shape[-1]
    return jnp.where(lane < size - 1, pltpu.roll(x, size - 1, x.ndim - 1), 0.0)


def _body(y_ref, a_ref, w_ref, o_ref):
    batch, m = y_ref.shape
    _, n = a_ref.shape
    f32 = jnp.float32

    a = a_ref[...]
    gram = lax.dot_general(a, a, (((0,), (0,)), ((), ())),
                           preferred_element_type=f32)
    ii = lax.broadcasted_iota(jnp.int32, (n, n), 0)
    jj = lax.broadcasted_iota(jnp.int32, (n, n), 1)
    mm = jnp.where(ii == jj, f32(1.0), f32(0.0)) - f32(_ETA) * gram

    y = y_ref[...]
    bt = f32(_ETA) * lax.dot_general(y, a, (((1,), (0,)), ((), ())),
                                     preferred_element_type=f32)
    w = w_ref[...]
    inv_tau = f32(1.0 / _TAU)

    xt = jnp.zeros((batch, n), dtype=f32)
    for layer in range(_N_SOFT_LAYERS):
        if layer == 0:
            ht = bt
        else:
            ht = bt + lax.dot_general(xt, mm, (((1,), (0,)), ((), ())),
                                      preferred_element_type=f32)
        s0 = jnp.abs(ht * w)

        t = _top128(s0.reshape(batch, n // _L, _L))
        t = t.reshape(batch, _L)
        t0 = t[:, 0:1]
        t63 = t[:, _S_TOPK - 1:_S_TOPK]

        lane = lax.broadcasted_iota(jnp.int32, (batch, _L), 1)
        msk64 = lane < _S_TOPK
        craw = jnp.maximum(jnp.exp((t - t0) * inv_tau), f32(1e-30))
        c = jnp.where(msk64, craw, f32(0.0))
        cinv = jnp.where(msk64, f32(1.0) / craw, f32(0.0))
        c63 = c[:, _S_TOPK - 1:_S_TOPK]

        w_j = jnp.exp((s0 - t0) * inv_tau)
        below = s0 < t63
        vw = jnp.sum(jnp.where(below, w_j, f32(0.0)), axis=1, keepdims=True)
        n_ge = jnp.sum(jnp.where(below, f32(0.0), f32(1.0)), axis=1,
                       keepdims=True)

        s_ge = _scan_sum(c, reverse=True)
        s_lt = _shift_right(_scan_sum(cinv, reverse=False), lane)
        z = (vw + (n_ge - f32(_S_TOPK)) * c63 + s_ge) * cinv + c * s_lt
        zinv = jnp.where(msk64, f32(1.0) / z, f32(0.0))
        alpha = zinv * cinv
        beta = zinv * c
        a_tot = jnp.sum(alpha, axis=1, keepdims=True)
        mval = (c * _scan_sum(alpha, reverse=False)
                + cinv * _shift_left(_scan_sum(beta, reverse=True), lane))

        m1 = jnp.where(lane < _L - 1, pltpu.roll(mval, _L - 1, 1), f32(0.0))
        dmv = jnp.where(lane < _S_TOPK - 1, m1 - mval, f32(0.0))
        dmv3 = dmv[:, :_S_TOPK, None]
        tcmp = t[:, :_S_TOPK, None]
        cbig = s0[:, None, :] < tcmp
        acc = mval[:, 0:1] + jnp.sum(
            jnp.where(cbig, dmv3, f32(0.0)), axis=1)
        mask = jnp.where(below, w_j * a_tot, acc)
        xt = mask * ht

    o_ref[...] = xt


def kernel(Y, A, W):
    batch, _ = Y.shape
    _, n = A.shape
    return pl.pallas_call(
        _body,
        out_shape=jax.ShapeDtypeStruct((batch, n), jnp.float32),
    )(Y, A, W.reshape(1, n))
